# baseline (speedup 1.0000x reference)
import functools

import jax
import jax.numpy as jnp
from jax.experimental import pallas as pl
from jax.experimental.pallas import tpu as pltpu

COMMITMENT_COST = 0.25

_TM = 512


def _body(nb, x_ref, cbt_ref, cb_ref, q_ref, idx_ref, dsum_ref, use_ref, uacc):
    x = x_ref[...]
    cbt = cbt_ref[...]
    sx = jnp.sum(x * x, axis=1, keepdims=True)
    se = jnp.sum(cbt * cbt, axis=0, keepdims=True)
    a2 = (2.0 * x).astype(jnp.bfloat16).astype(jnp.float32)
    m = jax.lax.dot_general(a2, cbt, (((1,), (0,)), ((), ())),
                            precision=jax.lax.Precision.HIGHEST,
                            preferred_element_type=jnp.float32)
    dist = (sx + se) - m
    mind = jnp.min(dist, axis=1, keepdims=True)
    iota = jax.lax.broadcasted_iota(jnp.int32, dist.shape, 1)
    idx = jnp.min(jnp.where(dist == mind, iota, jnp.int32(2**30)), axis=1)
    idx_ref[...] = idx

    onehot = (iota == idx[:, None]).astype(jnp.float32)
    q_ref[...] = jax.lax.dot_general(
        onehot, cb_ref[...], (((1,), (0,)), ((), ())),
        preferred_element_type=jnp.float32)

    @pl.when(pl.program_id(0) == 0)
    def _init():
        dsum_ref[...] = jnp.zeros((1, 1), jnp.float32)
        uacc[...] = jnp.zeros_like(uacc)

    dsum_ref[...] += jnp.sum(mind, axis=0, keepdims=True)
    uacc[...] += jnp.sum(onehot, axis=0, keepdims=True)

    @pl.when(pl.program_id(0) == nb - 1)
    def _fin():
        use_ref[...] = uacc[...].astype(jnp.int32)


def _vq(flat, codebook):
    n, d = flat.shape
    k = codebook.shape[0]
    nb = n // _TM
    return pl.pallas_call(
        functools.partial(_body, nb),
        grid=(nb,),
        in_specs=[
            pl.BlockSpec((_TM, d), lambda i: (i, 0)),
            pl.BlockSpec((d, k), lambda i: (0, 0)),
            pl.BlockSpec((k, d), lambda i: (0, 0)),
        ],
        out_specs=[
            pl.BlockSpec((_TM, d), lambda i: (i, 0)),
            pl.BlockSpec((_TM,), lambda i: (i,)),
            pl.BlockSpec((1, 1), lambda i: (0, 0)),
            pl.BlockSpec((1, k), lambda i: (0, 0)),
        ],
        out_shape=[
            jax.ShapeDtypeStruct((n, d), jnp.float32),
            jax.ShapeDtypeStruct((n,), jnp.int32),
            jax.ShapeDtypeStruct((1, 1), jnp.float32),
            jax.ShapeDtypeStruct((1, k), jnp.int32),
        ],
        scratch_shapes=[pltpu.VMEM((1, k), jnp.float32)],
    )(flat, codebook.T, codebook)


def kernel(z, codebook):
    b, t, d = z.shape
    k = codebook.shape[0]
    flat = z.reshape(-1, d)
    n = flat.shape[0]
    quantized, idx, dsum, usage = _vq(flat, codebook)
    vq_loss = ((1.0 + COMMITMENT_COST) / (n * d)) * dsum[0, 0]
    return (
        quantized.reshape(b, t, d),
        vq_loss,
        idx.reshape(b, t),
        usage.reshape(k),
    )
